# baseline (speedup 1.0000x reference)
import functools

import jax
import jax.numpy as jnp
from jax import lax
from jax.experimental import pallas as pl
from jax.experimental.pallas import tpu as pltpu
from jax.experimental.pallas import tpu_sc as plsc

B = 16384
NC = 2
NS = 16
NW = NC * NS
BPW = B // NW
CHUNK = 128
NCH = BPW // CHUNK

VC, DC = 1000000, 32
VI, DI = 100000, 48
DH, DD = 8, 8

CB_C = 8192
CB_I = 4096


def _relayout_client_body(wt_ref, out_ref):
    x = wt_ref[...]
    xt = jnp.transpose(x)
    x3 = xt.reshape(CB_C // 4, 4, 32)
    for p in range(4):
        out_ref[:, 32 * p:32 * (p + 1)] = x3[:, p, :]


def _relayout_client(wt):
    nb = (VC + CB_C - 1) // CB_C
    return pl.pallas_call(
        _relayout_client_body,
        grid=(nb,),
        in_specs=[pl.BlockSpec((DC, CB_C), lambda i: (0, i))],
        out_specs=pl.BlockSpec((CB_C // 4, 128), lambda i: (i, 0)),
        out_shape=jax.ShapeDtypeStruct((VC // 4, 128), jnp.float32),
    )(wt)


def _relayout_item_body(wt_ref, out_ref):
    x = wt_ref[...]
    xt = jnp.transpose(x)
    xp = jnp.pad(xt, ((0, 0), (0, 16)))
    x3 = xp.reshape(CB_I // 2, 2, 64)
    for p in range(2):
        out_ref[:, 64 * p:64 * (p + 1)] = x3[:, p, :]


def _relayout_item(wt):
    nb = (VI + CB_I - 1) // CB_I
    return pl.pallas_call(
        _relayout_item_body,
        grid=(nb,),
        in_specs=[pl.BlockSpec((DI, CB_I), lambda i: (0, i))],
        out_specs=pl.BlockSpec((CB_I // 2, 128), lambda i: (i, 0)),
        out_shape=jax.ShapeDtypeStruct((VI // 2, 128), jnp.float32),
    )(wt)


def _sc_gather_body(cidx_hbm, iidx_hbm, hidx_hbm, didx_hbm,
                    wc_hbm, wi_hbm, wh_hbm, wd_hbm,
                    out_c, out_i, out_h, out_d,
                    cidx_v, iidx_v, hidx_v, didx_v,
                    bufc, bufi, bufh, bufd,
                    sem):
    wid = lax.axis_index("s") * NC + lax.axis_index("c")
    base = wid * BPW
    pltpu.sync_copy(cidx_hbm.at[wid], cidx_v)
    pltpu.sync_copy(iidx_hbm.at[wid], iidx_v)
    pltpu.sync_copy(hidx_hbm.at[wid], hidx_v)
    pltpu.sync_copy(didx_hbm.at[wid], didx_v)

    def fire(j):
        p = j % 2
        return [
            pltpu.async_copy(wc_hbm.at[cidx_v.at[j]], bufc.at[p], sem),
            pltpu.async_copy(wi_hbm.at[iidx_v.at[j]], bufi.at[p], sem),
            pltpu.async_copy(wh_hbm.at[hidx_v.at[j]], bufh.at[p], sem),
            pltpu.async_copy(wd_hbm.at[didx_v.at[j]], bufd.at[p], sem),
        ]

    def drain(j, copies):
        p = j % 2
        for cp in copies:
            cp.wait()
        sl = pl.ds(base + j * CHUNK, CHUNK)
        pltpu.sync_copy(bufc.at[p], out_c.at[sl])
        pltpu.sync_copy(bufi.at[p], out_i.at[sl])
        pltpu.sync_copy(bufh.at[p], out_h.at[sl])
        pltpu.sync_copy(bufd.at[p], out_d.at[sl])

    pending = fire(0)
    for j in range(1, NCH):
        nxt = fire(j)
        drain(j - 1, pending)
        pending = nxt
    drain(NCH - 1, pending)


@functools.cache
def _sc_gather_kernel():
  return pl.kernel(
    _sc_gather_body,
    out_type=[
        jax.ShapeDtypeStruct((B, 128), jnp.float32),
        jax.ShapeDtypeStruct((B, 128), jnp.float32),
        jax.ShapeDtypeStruct((B, DH), jnp.float32),
        jax.ShapeDtypeStruct((B, DD), jnp.float32),
    ],
    mesh=plsc.VectorSubcoreMesh(
        core_axis_name="c", subcore_axis_name="s",
        num_cores=NC, num_subcores=NS),
    compiler_params=pltpu.CompilerParams(use_tc_tiling_on_sc=False),
    scratch_types=[
        pltpu.VMEM((NCH, CHUNK), jnp.int32),
        pltpu.VMEM((NCH, CHUNK), jnp.int32),
        pltpu.VMEM((NCH, CHUNK), jnp.int32),
        pltpu.VMEM((NCH, CHUNK), jnp.int32),
        pltpu.VMEM((2, CHUNK, 128), jnp.float32),
        pltpu.VMEM((2, CHUNK, 128), jnp.float32),
        pltpu.VMEM((2, CHUNK, DH), jnp.float32),
        pltpu.VMEM((2, CHUNK, DD), jnp.float32),
        pltpu.SemaphoreType.DMA,
    ],
  )


BLK = 2048
NB = B // BLK


def _mlp_body(dense, ecs, eis, eh, ed, m4, m2,
              wc, wh, wdw, wdu, bu1, wu2, bu2,
              wie, wdi, bi1, wi2, bi2,
              q_ref, c_ref):
    d = dense[...]
    f32 = jnp.float32
    m4v = m4[...]
    ec = ecs[:, 0:32] * m4v[:, 0:1]
    for p in range(1, 4):
        ec = ec + ecs[:, 32 * p:32 * (p + 1)] * m4v[:, p:p + 1]
    m2v = m2[...]
    ei = eis[:, 0:48] * m2v[:, 0:1]
    ei = ei + eis[:, 64:112] * m2v[:, 1:2]
    hu = jnp.dot(ec, wc[...], preferred_element_type=f32)
    hu = hu + jnp.dot(eh[...], wh[...], preferred_element_type=f32)
    hu = hu + jnp.dot(ed[...], wdw[...], preferred_element_type=f32)
    hu = hu + jnp.dot(d, wdu[...], preferred_element_type=f32)
    hu = jnp.maximum(hu + bu1[...], 0.0)
    q_ref[...] = jnp.maximum(
        jnp.dot(hu, wu2[...], preferred_element_type=f32) + bu2[...], 0.0)
    hi = jnp.dot(ei, wie[...], preferred_element_type=f32)
    hi = hi + jnp.dot(d, wdi[...], preferred_element_type=f32)
    hi = jnp.maximum(hi + bi1[...], 0.0)
    c_ref[...] = jnp.maximum(
        jnp.dot(hi, wi2[...], preferred_element_type=f32) + bi2[...], 0.0)


def _batch_spec(width):
    return pl.BlockSpec((BLK, width), lambda i: (i, 0))


def _full_spec(shape):
    return pl.BlockSpec(shape, lambda i: (0, 0))


def _mlp_call(dense, ecs, eis, eh, ed, m4, m2, weights):
    (wc, wh, wdw, wdu, bu1, wu2, bu2, wie, wdi, bi1, wi2, bi2) = weights
    in_specs = [
        _batch_spec(16), _batch_spec(128), _batch_spec(128),
        _batch_spec(DH), _batch_spec(DD), _batch_spec(4), _batch_spec(2),
        _full_spec(wc.shape), _full_spec(wh.shape), _full_spec(wdw.shape),
        _full_spec(wdu.shape), _full_spec(bu1.shape), _full_spec(wu2.shape),
        _full_spec(bu2.shape), _full_spec(wie.shape), _full_spec(wdi.shape),
        _full_spec(bi1.shape), _full_spec(wi2.shape), _full_spec(bi2.shape),
    ]
    out_specs = [_batch_spec(32), _batch_spec(32)]
    return pl.pallas_call(
        _mlp_body,
        grid=(NB,),
        in_specs=in_specs,
        out_specs=out_specs,
        out_shape=[
            jax.ShapeDtypeStruct((B, 32), jnp.float32),
            jax.ShapeDtypeStruct((B, 32), jnp.float32),
        ],
    )(dense, ecs, eis, eh, ed, m4, m2, wc, wh, wdw, wdu, bu1, wu2, bu2,
      wie, wdi, bi1, wi2, bi2)


def kernel(dense_features, client_index, hour, dayOfWeek, item_index,
           W_client, W_hour, W_dow, W_item,
           Wu1, bu1, Wu2, bu2, Wi1, bi1, Wi2, bi2):
    cidx = client_index.astype(jnp.int32)
    iidx = item_index.astype(jnp.int32)
    cidx4 = (cidx // 4).reshape(NW, NCH, CHUNK)
    iidx2 = (iidx // 2).reshape(NW, NCH, CHUNK)
    hidx = hour.astype(jnp.int32).reshape(NW, NCH, CHUNK)
    didx = dayOfWeek.astype(jnp.int32).reshape(NW, NCH, CHUNK)
    m4 = (cidx[:, None] % 4 == jnp.arange(4)[None, :]).astype(jnp.float32)
    m2 = (iidx[:, None] % 2 == jnp.arange(2)[None, :]).astype(jnp.float32)

    wc_packed = _relayout_client(W_client.T)
    wi_packed = _relayout_item(W_item.T)
    w_dow_p = jnp.pad(W_dow, ((0, 0), (0, DD - 4)))

    ecs, eis, eh, ed = _sc_gather_kernel()(cidx4, iidx2, hidx, didx,
                                           wc_packed, wi_packed,
                                           W_hour, w_dow_p)

    wc = Wu1[0:32]
    wh = Wu1[32:40]
    wdw = jnp.concatenate([Wu1[40:44], jnp.zeros((DD - 4, 64), jnp.float32)],
                          axis=0)
    wdu = jnp.concatenate([Wu1[44:53], jnp.zeros((7, 64), jnp.float32)], axis=0)
    wie = Wi1[0:48]
    wdi = jnp.concatenate([jnp.zeros((9, 64), jnp.float32), Wi1[48:55]], axis=0)

    weights = (wc, wh, wdw, wdu, bu1.reshape(1, 64), Wu2, bu2.reshape(1, 32),
               wie, wdi, bi1.reshape(1, 64), Wi2, bi2.reshape(1, 32))
    q, c = _mlp_call(dense_features, ecs, eis, eh, ed, m4, m2, weights)
    return (q, c)


# speedup vs baseline: 1.0652x; 1.0652x over previous
import functools

import jax
import jax.numpy as jnp
from jax import lax
from jax.experimental import pallas as pl
from jax.experimental.pallas import tpu as pltpu
from jax.experimental.pallas import tpu_sc as plsc

B = 16384
NC = 2
NS = 16
NW = NC * NS
BPW = B // NW
CHUNK = 128
NCH = BPW // CHUNK

VC, DC = 1000000, 32
VI, DI = 100000, 48
DH, DD = 8, 8

CB_C = 8192
CB_I = 4096


def _relayout_client_body(wt_ref, out_ref):
    x = wt_ref[...]
    eye = (jax.lax.broadcasted_iota(jnp.int32, (DC, DC), 0) ==
           jax.lax.broadcasted_iota(jnp.int32, (DC, DC), 1)).astype(jnp.float32)
    xt = jax.lax.dot_general(x, eye, (((0,), (0,)), ((), ())),
                             preferred_element_type=jnp.float32)
    x3 = xt.reshape(CB_C // 4, 4, 32)
    for p in range(4):
        out_ref[:, 32 * p:32 * (p + 1)] = x3[:, p, :]


def _relayout_client(wt):
    nb = (VC + CB_C - 1) // CB_C
    return pl.pallas_call(
        _relayout_client_body,
        grid=(nb,),
        in_specs=[pl.BlockSpec((DC, CB_C), lambda i: (0, i))],
        out_specs=pl.BlockSpec((CB_C // 4, 128), lambda i: (i, 0)),
        out_shape=jax.ShapeDtypeStruct((VC // 4, 128), jnp.float32),
    )(wt)


def _relayout_item_body(wt_ref, out_ref):
    x = wt_ref[...]
    eyep = (jax.lax.broadcasted_iota(jnp.int32, (DI, 64), 0) ==
            jax.lax.broadcasted_iota(jnp.int32, (DI, 64), 1)).astype(jnp.float32)
    xp = jax.lax.dot_general(x, eyep, (((0,), (0,)), ((), ())),
                             preferred_element_type=jnp.float32)
    x3 = xp.reshape(CB_I // 2, 2, 64)
    for p in range(2):
        out_ref[:, 64 * p:64 * (p + 1)] = x3[:, p, :]


def _relayout_item(wt):
    nb = (VI + CB_I - 1) // CB_I
    return pl.pallas_call(
        _relayout_item_body,
        grid=(nb,),
        in_specs=[pl.BlockSpec((DI, CB_I), lambda i: (0, i))],
        out_specs=pl.BlockSpec((CB_I // 2, 128), lambda i: (i, 0)),
        out_shape=jax.ShapeDtypeStruct((VI // 2, 128), jnp.float32),
    )(wt)


def _sc_gather_body(cidx_hbm, iidx_hbm,
                    wc_hbm, wi_hbm,
                    out_c, out_i,
                    cidx_v, iidx_v,
                    bufc, bufi,
                    sem):
    wid = lax.axis_index("s") * NC + lax.axis_index("c")
    base = wid * BPW
    pltpu.sync_copy(cidx_hbm.at[wid], cidx_v)
    pltpu.sync_copy(iidx_hbm.at[wid], iidx_v)

    def fire(j):
        p = j % 2
        return [
            pltpu.async_copy(wc_hbm.at[cidx_v.at[j]], bufc.at[p], sem),
            pltpu.async_copy(wi_hbm.at[iidx_v.at[j]], bufi.at[p], sem),
        ]

    def drain(j, copies):
        p = j % 2
        for cp in copies:
            cp.wait()
        sl = pl.ds(base + j * CHUNK, CHUNK)
        pltpu.sync_copy(bufc.at[p], out_c.at[sl])
        pltpu.sync_copy(bufi.at[p], out_i.at[sl])

    pending = fire(0)
    for j in range(1, NCH):
        nxt = fire(j)
        drain(j - 1, pending)
        pending = nxt
    drain(NCH - 1, pending)


@functools.cache
def _sc_gather_kernel():
  return pl.kernel(
    _sc_gather_body,
    out_type=[
        jax.ShapeDtypeStruct((B, 128), jnp.float32),
        jax.ShapeDtypeStruct((B, 128), jnp.float32),
    ],
    mesh=plsc.VectorSubcoreMesh(
        core_axis_name="c", subcore_axis_name="s",
        num_cores=NC, num_subcores=NS),
    compiler_params=pltpu.CompilerParams(use_tc_tiling_on_sc=False),
    scratch_types=[
        pltpu.VMEM((NCH, CHUNK), jnp.int32),
        pltpu.VMEM((NCH, CHUNK), jnp.int32),
        pltpu.VMEM((2, CHUNK, 128), jnp.float32),
        pltpu.VMEM((2, CHUNK, 128), jnp.float32),
        pltpu.SemaphoreType.DMA,
    ],
  )


BLK = 2048
NB = B // BLK


def _mlp_body(dense, ecs, eis, m4, m2, m24, m7,
              wch, tbl_h, wh, tbl_d, wdw, wdu, bu1, wu2, bu2,
              wie, wdi, bi1, wi2, bi2,
              q_ref, c_ref):
    d = dense[...]
    f32 = jnp.float32
    m4v = m4[...]
    ec = ecs[:, 0:32] * m4v[:, 0:1]
    for p in range(1, 4):
        ec = ec + ecs[:, 32 * p:32 * (p + 1)] * m4v[:, p:p + 1]
    m2v = m2[...]
    ei = eis[:, 0:48] * m2v[:, 0:1]
    ei = ei + eis[:, 64:112] * m2v[:, 1:2]
    eh = jnp.dot(m24[...], tbl_h[...], preferred_element_type=f32)
    ed = jnp.dot(m7[...], tbl_d[...], preferred_element_type=f32)
    hu = jnp.dot(ec, wch[...], preferred_element_type=f32)
    hu = hu + jnp.dot(eh, wh[...], preferred_element_type=f32)
    hu = hu + jnp.dot(ed, wdw[...], preferred_element_type=f32)
    hu = hu + jnp.dot(d, wdu[...], preferred_element_type=f32)
    hu = jnp.maximum(hu + bu1[...], 0.0)
    q_ref[...] = jnp.maximum(
        jnp.dot(hu, wu2[...], preferred_element_type=f32) + bu2[...], 0.0)
    hi = jnp.dot(ei, wie[...], preferred_element_type=f32)
    hi = hi + jnp.dot(d, wdi[...], preferred_element_type=f32)
    hi = jnp.maximum(hi + bi1[...], 0.0)
    c_ref[...] = jnp.maximum(
        jnp.dot(hi, wi2[...], preferred_element_type=f32) + bi2[...], 0.0)


def _batch_spec(width):
    return pl.BlockSpec((BLK, width), lambda i: (i, 0))


def _full_spec(shape):
    return pl.BlockSpec(shape, lambda i: (0, 0))


def _mlp_call(dense, ecs, eis, m4, m2, m24, m7, weights):
    (wch, tbl_h, wh, tbl_d, wdw, wdu, bu1, wu2, bu2,
     wie, wdi, bi1, wi2, bi2) = weights
    batch_args = (dense, ecs, eis, m4, m2, m24, m7)
    in_specs = [_batch_spec(a.shape[1]) for a in batch_args]
    in_specs += [_full_spec(w.shape) for w in weights]
    out_specs = [_batch_spec(32), _batch_spec(32)]
    return pl.pallas_call(
        _mlp_body,
        grid=(NB,),
        in_specs=in_specs,
        out_specs=out_specs,
        out_shape=[
            jax.ShapeDtypeStruct((B, 32), jnp.float32),
            jax.ShapeDtypeStruct((B, 32), jnp.float32),
        ],
    )(*batch_args, *weights)


def kernel(dense_features, client_index, hour, dayOfWeek, item_index,
           W_client, W_hour, W_dow, W_item,
           Wu1, bu1, Wu2, bu2, Wi1, bi1, Wi2, bi2):
    cidx = client_index.astype(jnp.int32)
    iidx = item_index.astype(jnp.int32)
    cidx4 = (cidx // 4).reshape(NW, NCH, CHUNK)
    iidx2 = (iidx // 2).reshape(NW, NCH, CHUNK)
    m4 = (cidx[:, None] % 4 == jnp.arange(4)[None, :]).astype(jnp.float32)
    m2 = (iidx[:, None] % 2 == jnp.arange(2)[None, :]).astype(jnp.float32)
    m24 = (hour[:, None] == jnp.arange(24)[None, :]).astype(jnp.float32)
    m7 = (dayOfWeek[:, None] == jnp.arange(7)[None, :]).astype(jnp.float32)

    wc_packed = _relayout_client(W_client.T)
    wi_packed = _relayout_item(W_item.T)

    ecs, eis = _sc_gather_kernel()(cidx4, iidx2, wc_packed, wi_packed)

    wch = Wu1[0:32]
    wh = Wu1[32:40]
    wdw = Wu1[40:44]
    wdu = jnp.concatenate([Wu1[44:53], jnp.zeros((7, 64), jnp.float32)], axis=0)
    wie = Wi1[0:48]
    wdi = jnp.concatenate([jnp.zeros((9, 64), jnp.float32), Wi1[48:55]], axis=0)

    weights = (wch, W_hour, wh, W_dow, wdw, wdu, bu1.reshape(1, 64), Wu2,
               bu2.reshape(1, 32), wie, wdi, bi1.reshape(1, 64), Wi2,
               bi2.reshape(1, 32))
    q, c = _mlp_call(dense_features, ecs, eis, m4, m2, m24, m7, weights)
    return (q, c)


# speedup vs baseline: 1.1235x; 1.0548x over previous
import functools

import jax
import jax.numpy as jnp
from jax import lax
from jax.experimental import pallas as pl
from jax.experimental.pallas import tpu as pltpu
from jax.experimental.pallas import tpu_sc as plsc

B = 16384
NC = 2
NS = 16
NW = NC * NS
BPW = B // NW
CHUNK = 128
NCH = BPW // CHUNK

VC, DC = 1000000, 32
VI, DI = 100000, 48
DH, DD = 8, 8

CB_C = 8192
CB_I = 4096


def _relayout_client_body(wt_ref, out_ref):
    x = wt_ref[...]
    eye = (jax.lax.broadcasted_iota(jnp.int32, (DC, DC), 0) ==
           jax.lax.broadcasted_iota(jnp.int32, (DC, DC), 1)).astype(jnp.float32)
    xt = jnp.dot(jnp.transpose(x), eye,
                 preferred_element_type=jnp.float32)
    x3 = xt.reshape(CB_C // 4, 4, 32)
    for p in range(4):
        out_ref[:, 32 * p:32 * (p + 1)] = x3[:, p, :]


def _relayout_client(wt):
    nb = (VC + CB_C - 1) // CB_C
    return pl.pallas_call(
        _relayout_client_body,
        grid=(nb,),
        in_specs=[pl.BlockSpec((DC, CB_C), lambda i: (0, i))],
        out_specs=pl.BlockSpec((CB_C // 4, 128), lambda i: (i, 0)),
        out_shape=jax.ShapeDtypeStruct((VC // 4, 128), jnp.float32),
        compiler_params=pltpu.CompilerParams(
            fuse_transposed_lhs_in_matmul=True),
    )(wt)


def _relayout_item_body(wt_ref, out_ref):
    x = wt_ref[...]
    eyep = (jax.lax.broadcasted_iota(jnp.int32, (DI, 64), 0) ==
            jax.lax.broadcasted_iota(jnp.int32, (DI, 64), 1)).astype(jnp.float32)
    xp = jnp.dot(jnp.transpose(x), eyep,
                 preferred_element_type=jnp.float32)
    x3 = xp.reshape(CB_I // 2, 2, 64)
    for p in range(2):
        out_ref[:, 64 * p:64 * (p + 1)] = x3[:, p, :]


def _relayout_item(wt):
    nb = (VI + CB_I - 1) // CB_I
    return pl.pallas_call(
        _relayout_item_body,
        grid=(nb,),
        in_specs=[pl.BlockSpec((DI, CB_I), lambda i: (0, i))],
        out_specs=pl.BlockSpec((CB_I // 2, 128), lambda i: (i, 0)),
        out_shape=jax.ShapeDtypeStruct((VI // 2, 128), jnp.float32),
        compiler_params=pltpu.CompilerParams(
            fuse_transposed_lhs_in_matmul=True),
    )(wt)


def _sc_gather_body(cidx_hbm, iidx_hbm,
                    wc_hbm, wi_hbm,
                    out_c, out_i,
                    cidx_v, iidx_v,
                    bufc, bufi,
                    sem):
    wid = lax.axis_index("s") * NC + lax.axis_index("c")
    base = wid * BPW
    pltpu.sync_copy(cidx_hbm.at[wid], cidx_v)
    pltpu.sync_copy(iidx_hbm.at[wid], iidx_v)

    def fire(j):
        p = j % 2
        return [
            pltpu.async_copy(wc_hbm.at[cidx_v.at[j]], bufc.at[p], sem),
            pltpu.async_copy(wi_hbm.at[iidx_v.at[j]], bufi.at[p], sem),
        ]

    def drain(j, copies):
        p = j % 2
        for cp in copies:
            cp.wait()
        sl = pl.ds(base + j * CHUNK, CHUNK)
        pltpu.sync_copy(bufc.at[p], out_c.at[sl])
        pltpu.sync_copy(bufi.at[p], out_i.at[sl])

    pending = fire(0)
    for j in range(1, NCH):
        nxt = fire(j)
        drain(j - 1, pending)
        pending = nxt
    drain(NCH - 1, pending)


@functools.cache
def _sc_gather_kernel():
  return pl.kernel(
    _sc_gather_body,
    out_type=[
        jax.ShapeDtypeStruct((B, 128), jnp.float32),
        jax.ShapeDtypeStruct((B, 128), jnp.float32),
    ],
    mesh=plsc.VectorSubcoreMesh(
        core_axis_name="c", subcore_axis_name="s",
        num_cores=NC, num_subcores=NS),
    compiler_params=pltpu.CompilerParams(use_tc_tiling_on_sc=False),
    scratch_types=[
        pltpu.VMEM((NCH, CHUNK), jnp.int32),
        pltpu.VMEM((NCH, CHUNK), jnp.int32),
        pltpu.VMEM((2, CHUNK, 128), jnp.float32),
        pltpu.VMEM((2, CHUNK, 128), jnp.float32),
        pltpu.SemaphoreType.DMA,
    ],
  )


BLK = 2048
NB = B // BLK


def _mlp_body(dense, ecs, eis, masks,
              wch, tbl_h, wh, tbl_d, wdw, wdu, bu1, wu2, bu2,
              wie, wdi, bi1, wi2, bi2,
              q_ref, c_ref):
    d = dense[...]
    f32 = jnp.float32
    mv = masks[...]
    ec = ecs[:, 0:32] * mv[:, 0:1]
    for p in range(1, 4):
        ec = ec + ecs[:, 32 * p:32 * (p + 1)] * mv[:, p:p + 1]
    ei = eis[:, 0:48] * mv[:, 8:9]
    ei = ei + eis[:, 64:112] * mv[:, 9:10]
    eh = jnp.dot(mv[:, 16:40], tbl_h[...], preferred_element_type=f32)
    ed = jnp.dot(mv[:, 40:47], tbl_d[...], preferred_element_type=f32)
    hu = jnp.dot(ec, wch[...], preferred_element_type=f32)
    hu = hu + jnp.dot(eh, wh[...], preferred_element_type=f32)
    hu = hu + jnp.dot(ed, wdw[...], preferred_element_type=f32)
    hu = hu + jnp.dot(d, wdu[...], preferred_element_type=f32)
    hu = jnp.maximum(hu + bu1[...], 0.0)
    q_ref[...] = jnp.maximum(
        jnp.dot(hu, wu2[...], preferred_element_type=f32) + bu2[...], 0.0)
    hi = jnp.dot(ei, wie[...], preferred_element_type=f32)
    hi = hi + jnp.dot(d, wdi[...], preferred_element_type=f32)
    hi = jnp.maximum(hi + bi1[...], 0.0)
    c_ref[...] = jnp.maximum(
        jnp.dot(hi, wi2[...], preferred_element_type=f32) + bi2[...], 0.0)


def _batch_spec(width):
    return pl.BlockSpec((BLK, width), lambda i: (i, 0))


def _full_spec(shape):
    return pl.BlockSpec(shape, lambda i: (0, 0))


def _mlp_call(dense, ecs, eis, masks, weights):
    (wch, tbl_h, wh, tbl_d, wdw, wdu, bu1, wu2, bu2,
     wie, wdi, bi1, wi2, bi2) = weights
    batch_args = (dense, ecs, eis, masks)
    in_specs = [_batch_spec(a.shape[1]) for a in batch_args]
    in_specs += [_full_spec(w.shape) for w in weights]
    out_specs = [_batch_spec(32), _batch_spec(32)]
    return pl.pallas_call(
        _mlp_body,
        grid=(NB,),
        in_specs=in_specs,
        out_specs=out_specs,
        out_shape=[
            jax.ShapeDtypeStruct((B, 32), jnp.float32),
            jax.ShapeDtypeStruct((B, 32), jnp.float32),
        ],
    )(*batch_args, *weights)


def kernel(dense_features, client_index, hour, dayOfWeek, item_index,
           W_client, W_hour, W_dow, W_item,
           Wu1, bu1, Wu2, bu2, Wi1, bi1, Wi2, bi2):
    cidx = client_index.astype(jnp.int32)
    iidx = item_index.astype(jnp.int32)
    cidx4 = (cidx // 4).reshape(NW, NCH, CHUNK)
    iidx2 = (iidx // 2).reshape(NW, NCH, CHUNK)
    lanes = jnp.arange(64)[None, :]
    masks = ((cidx[:, None] % 4 + 0 == lanes) |
             (iidx[:, None] % 2 + 8 == lanes) |
             (hour[:, None].astype(jnp.int32) + 16 == lanes) |
             (dayOfWeek[:, None].astype(jnp.int32) + 40 == lanes)
             ).astype(jnp.float32)

    wc_packed = _relayout_client(W_client.T)
    wi_packed = _relayout_item(W_item.T)

    ecs, eis = _sc_gather_kernel()(cidx4, iidx2, wc_packed, wi_packed)

    wch = Wu1[0:32]
    wh = Wu1[32:40]
    wdw = Wu1[40:44]
    wdu = jnp.concatenate([Wu1[44:53], jnp.zeros((7, 64), jnp.float32)], axis=0)
    wie = Wi1[0:48]
    wdi = jnp.concatenate([jnp.zeros((9, 64), jnp.float32), Wi1[48:55]], axis=0)

    weights = (wch, W_hour, wh, W_dow, wdw, wdu, bu1.reshape(1, 64), Wu2,
               bu2.reshape(1, 32), wie, wdi, bi1.reshape(1, 64), Wi2,
               bi2.reshape(1, 32))
    q, c = _mlp_call(dense_features, ecs, eis, masks, weights)
    return (q, c)
